# baseline (speedup 1.0000x reference)
import functools
import jax
import jax.numpy as jnp
from jax.experimental import pallas as pl
from jax.experimental.pallas import tpu as pltpu

_ENH = 32


def _update_body(agg_ref, cnt_ref, x_ref, r_ref, b_ref, o_ref, *, mean, act):
    agg = agg_ref[...]
    if mean:
        agg = agg / jnp.maximum(cnt_ref[...], 1.0)
    y = agg + jnp.dot(x_ref[...], r_ref[...], preferred_element_type=jnp.float32)
    y = y + b_ref[...]
    if act:
        y = jnp.maximum(y, 0.0)
    o_ref[...] = y


def _node_update(agg, cnt, x_in, root, bias, mean, act):
    n, dout = agg.shape
    return pl.pallas_call(
        functools.partial(_update_body, mean=mean, act=act),
        out_shape=jax.ShapeDtypeStruct((n, dout), jnp.float32),
    )(agg, cnt, x_in, root, bias.reshape(1, dout))


def _layer(x_in, src, dst, ea, p, cnt, mean, act):
    din, dout = p['root'].shape
    w2t = p['w2'].reshape(_ENH, din, dout).transpose(1, 0, 2).reshape(din, _ENH * dout)
    tab = x_in @ w2t
    c = x_in @ p['b2'].reshape(din, dout)
    h = jax.nn.relu(ea @ p['w1'] + p['b1'])
    msg = jnp.einsum('ek,eko->eo', h, tab[src].reshape(-1, _ENH, dout)) + c[src]
    agg = jax.ops.segment_sum(msg, dst, num_segments=x_in.shape[0])
    return _node_update(agg, cnt, x_in, p['root'], p['bias'], mean, act)


def kernel(x, edge_index, edge_attr, params):
    src = edge_index[0]
    dst = edge_index[1]
    n = x.shape[0]
    cnt = jax.ops.segment_sum(
        jnp.ones((dst.shape[0],), jnp.float32), dst, num_segments=n)[:, None]
    h1 = _layer(x, src, dst, edge_attr, params['conv1'], cnt, True, True)
    h2 = _layer(h1, src, dst, edge_attr, params['conv2'], cnt, True, True)
    mu = _layer(h2, src, dst, edge_attr, params['mu'], cnt, False, False)
    lv = _layer(h2, src, dst, edge_attr, params['logvar'], cnt, False, False)
    return (mu, lv)


# speedup vs baseline: 1.5588x; 1.5588x over previous
import functools
import jax
import jax.numpy as jnp
from jax import lax
from jax.experimental import pallas as pl
from jax.experimental.pallas import tpu as pltpu
from jax.experimental.pallas import tpu_sc as plsc

_ENH = 32
_NC = 2
_NS = 16
_NW = _NC * _NS
_C = 64
_TW = 256
_HW = 48


def _edge_body(with_cnt, tab_h, h_h, src_h, dst_h, *rest):
    if with_cnt:
        parts_h, cnts_h, idx_v, dst_v, h_v, rows_v, accT_v, cnt_v, sem = rest
    else:
        parts_h, idx_v, dst_v, h_v, rows_v, accT_v, sem = rest
        cnt_v = None
    cid = lax.axis_index("c")
    sid = lax.axis_index("s")
    wid = sid * _NC + cid

    e_total = src_h.shape[0]
    nchunks = e_total // _C
    base_chunks = nchunks // _NW
    extra = nchunks - base_chunks * _NW
    n_nodes = accT_v.shape[1]
    ngroups = n_nodes // 16

    lane = lax.iota(jnp.int32, 16)
    half = lax.shift_right_logical(lane, 3)
    pats = [half + (2 * m) for m in range(8)]
    bc = [lane * 0 + k for k in range(16)]
    fold = (lane & 7) + 8
    rowv = lane & 7
    low8 = lane < 8
    lane0 = lane == 0
    zv = jnp.zeros((16,), jnp.float32)
    onev = zv + 1.0

    def zrow(r):
        def zb(l, carry):
            accT_v[r, pl.ds(l * 16, 16)] = zv
            return carry
        lax.fori_loop(0, ngroups, zb, 0)
    for r in range(8):
        zrow(r)
    if with_cnt:
        def zc(l, carry):
            cnt_v[pl.ds(l * 16, 16)] = zv
            return carry
        lax.fori_loop(0, ngroups, zc, 0)

    def do_chunk(chunk_id):
        ebase = chunk_id * _C
        pltpu.sync_copy(src_h.at[pl.ds(ebase, _C)], idx_v)
        pltpu.sync_copy(dst_h.at[pl.ds(ebase, _C)], dst_v)
        pltpu.sync_copy(h_h.at[pl.ds(ebase, _C), :], h_v)
        pltpu.async_copy(tab_h.at[idx_v], rows_v, sem).wait()

        def group(g, carry):
            dwin = dst_v[pl.ds(g * 16, 16)]
            for k in range(16):
                e = g * 16 + k
                h_a = h_v[e, pl.ds(0, 16)]
                h_b = h_v[e, pl.ds(16, 16)]
                acc = h_v[e, pl.ds(32, 16)]
                for j in range(16):
                    hs = h_a if j < 8 else h_b
                    hj = hs.at[pats[j % 8]].get(mode='promise_in_bounds')
                    acc = acc + hj * rows_v[e, pl.ds(j * 16, 16)]
                accf = acc.at[fold].get(mode='promise_in_bounds')
                m = acc + accf
                colv = dwin.at[bc[k]].get(mode='promise_in_bounds')
                plsc.addupdate_scatter(accT_v, [rowv, colv], m, mask=low8)
                if with_cnt:
                    plsc.addupdate_scatter(cnt_v, [colv], onev, mask=lane0)
            return carry

        lax.fori_loop(0, _C // 16, group, 0)

    def chunk_loop(q, carry):
        do_chunk(wid + q * _NW)
        return carry

    lax.fori_loop(0, base_chunks, chunk_loop, 0)

    @pl.when(wid < extra)
    def _extra():
        do_chunk(base_chunks * _NW + wid)

    pltpu.sync_copy(accT_v, parts_h.at[wid])
    if with_cnt:
        pltpu.sync_copy(cnt_v, cnts_h.at[wid])


def _sc_edge_phase(tab, h, src, dst, with_cnt):
    n = tab.shape[0]
    mesh = plsc.VectorSubcoreMesh(core_axis_name="c", subcore_axis_name="s",
                                  num_cores=_NC, num_subcores=_NS)
    out_type = [jax.ShapeDtypeStruct((_NW, 8, n), jnp.float32)]
    scratch = [
        pltpu.VMEM((_C,), jnp.int32),
        pltpu.VMEM((_C,), jnp.int32),
        pltpu.VMEM((_C, _HW), jnp.float32),
        pltpu.VMEM((_C, _TW), jnp.float32),
        pltpu.VMEM((8, n), jnp.float32),
    ]
    if with_cnt:
        out_type.append(jax.ShapeDtypeStruct((_NW, n), jnp.float32))
        scratch.append(pltpu.VMEM((n,), jnp.float32))
    scratch.append(pltpu.SemaphoreType.DMA)
    return pl.kernel(
        functools.partial(_edge_body, with_cnt),
        out_type=tuple(out_type),
        mesh=mesh,
        scratch_types=scratch,
        compiler_params=pltpu.CompilerParams(needs_layout_passes=False),
    )(tab, h, src, dst)


def _update_body(*refs, mean, act):
    if mean:
        parts_ref, cnts_ref, xT_ref, rT_ref, b_ref, o_ref = refs
    else:
        parts_ref, xT_ref, rT_ref, b_ref, o_ref = refs
    aggT = jnp.sum(parts_ref[...], axis=0)
    if mean:
        cnt = jnp.sum(cnts_ref[...], axis=0)[None, :]
        aggT = aggT / jnp.maximum(cnt, 1.0)
    y = aggT + jnp.dot(rT_ref[...], xT_ref[...],
                       preferred_element_type=jnp.float32)
    y = y + b_ref[...]
    if act:
        y = jnp.maximum(y, 0.0)
    o_ref[...] = y


def _node_update(parts, cnts, xT, root, bias, mean, act):
    n = xT.shape[1]
    dout = root.shape[1]
    args = [parts] + ([cnts] if mean else []) + [
        xT, root.T, bias.reshape(dout, 1)]
    return pl.pallas_call(
        functools.partial(_update_body, mean=mean, act=act),
        out_shape=jax.ShapeDtypeStruct((dout, n), jnp.float32),
    )(*args)


def _layer(x_in, src, dst, h_edge, p, cnts, mean, act):
    din, dout = p['root'].shape
    w2t = p['w2'].reshape(_ENH, din, dout).transpose(1, 0, 2).reshape(din, _ENH * dout)
    tab = x_in @ w2t
    c = x_in @ p['b2'].reshape(din, dout)
    e = src.shape[0]
    hfull = jnp.concatenate(
        [h_edge, c[src], jnp.zeros((e, _HW - _ENH - dout), jnp.float32)], axis=1)
    if cnts is None and mean:
        parts, cnts = _sc_edge_phase(tab, hfull, src, dst, True)
    else:
        (parts,) = _sc_edge_phase(tab, hfull, src, dst, False)
    yT = _node_update(parts, cnts, x_in.T, p['root'], p['bias'], mean, act)
    return yT.T, cnts


def kernel(x, edge_index, edge_attr, params):
    src = edge_index[0]
    dst = edge_index[1]
    h1e = jax.nn.relu(edge_attr @ params['conv1']['w1'] + params['conv1']['b1'])
    h2e = jax.nn.relu(edge_attr @ params['conv2']['w1'] + params['conv2']['b1'])
    hme = jax.nn.relu(edge_attr @ params['mu']['w1'] + params['mu']['b1'])
    hle = jax.nn.relu(edge_attr @ params['logvar']['w1'] + params['logvar']['b1'])
    h1, cnts = _layer(x, src, dst, h1e, params['conv1'], None, True, True)
    h2, _ = _layer(h1, src, dst, h2e, params['conv2'], cnts, True, True)
    mu, _ = _layer(h2, src, dst, hme, params['mu'], None, False, False)
    lv, _ = _layer(h2, src, dst, hle, params['logvar'], None, False, False)
    return (mu, lv)


# speedup vs baseline: 1.6134x; 1.0350x over previous
import functools
import jax
import jax.numpy as jnp
from jax import lax
from jax.experimental import pallas as pl
from jax.experimental.pallas import tpu as pltpu
from jax.experimental.pallas import tpu_sc as plsc

_ENH = 32
_NC = 2
_NS = 16
_NW = _NC * _NS
_C = 64
_TW = 256
_HW = 48


def _edge_body(with_cnt, tab_h, h_h, src_h, dst_h, *rest):
    if with_cnt:
        parts_h, cnts_h, idx_v, dst_v, h_v, rows_v, accT_v, cnt_v, sem = rest
    else:
        parts_h, idx_v, dst_v, h_v, rows_v, accT_v, sem = rest
        cnt_v = None
    cid = lax.axis_index("c")
    sid = lax.axis_index("s")
    wid = sid * _NC + cid

    e_total = src_h.shape[0]
    nchunks = e_total // _C
    base_chunks = nchunks // _NW
    extra = nchunks - base_chunks * _NW
    n_nodes = accT_v.shape[1]
    ngroups = n_nodes // 16

    lane = lax.iota(jnp.int32, 16)
    half = lax.shift_right_logical(lane, 3)
    pats = [half + (2 * m) for m in range(8)]
    bc = [lane * 0 + k for k in range(16)]
    fold = (lane & 7) + 8
    rowv = lane & 7
    low8 = lane < 8
    lane0 = lane == 0
    zv = jnp.zeros((16,), jnp.float32)
    onev = zv + 1.0

    def zrow(r):
        def zb(l, carry):
            accT_v[r, pl.ds(l * 16, 16)] = zv
            return carry
        lax.fori_loop(0, ngroups, zb, 0)
    for r in range(8):
        zrow(r)
    if with_cnt:
        def zc(l, carry):
            cnt_v[pl.ds(l * 16, 16)] = zv
            return carry
        lax.fori_loop(0, ngroups, zc, 0)

    def do_chunk(chunk_id):
        ebase = chunk_id * _C
        pltpu.sync_copy(src_h.at[pl.ds(ebase, _C)], idx_v)
        pltpu.sync_copy(dst_h.at[pl.ds(ebase, _C)], dst_v)
        pltpu.sync_copy(h_h.at[pl.ds(ebase, _C), :], h_v)
        pltpu.async_copy(tab_h.at[idx_v], rows_v, sem).wait()

        def group(g, carry):
            dwin = dst_v[pl.ds(g * 16, 16)]
            for k in range(16):
                e = g * 16 + k
                h_a = h_v[e, pl.ds(0, 16)]
                h_b = h_v[e, pl.ds(16, 16)]
                accs = [h_v[e, pl.ds(32, 16)], zv, zv, zv]
                for j in range(16):
                    hs = h_a if j < 8 else h_b
                    hj = hs.at[pats[j % 8]].get(mode='promise_in_bounds')
                    accs[j % 4] = accs[j % 4] + hj * rows_v[e, pl.ds(j * 16, 16)]
                acc = (accs[0] + accs[1]) + (accs[2] + accs[3])
                accf = acc.at[fold].get(mode='promise_in_bounds')
                m = acc + accf
                colv = dwin.at[bc[k]].get(mode='promise_in_bounds')
                plsc.addupdate_scatter(accT_v, [rowv, colv], m, mask=low8)
                if with_cnt:
                    plsc.addupdate_scatter(cnt_v, [colv], onev, mask=lane0)
            return carry

        lax.fori_loop(0, _C // 16, group, 0)

    def chunk_loop(q, carry):
        do_chunk(wid + q * _NW)
        return carry

    lax.fori_loop(0, base_chunks, chunk_loop, 0)

    @pl.when(wid < extra)
    def _extra():
        do_chunk(base_chunks * _NW + wid)

    pltpu.sync_copy(accT_v, parts_h.at[wid])
    if with_cnt:
        pltpu.sync_copy(cnt_v, cnts_h.at[wid])


def _sc_edge_phase(tab, h, src, dst, with_cnt):
    n = tab.shape[0]
    mesh = plsc.VectorSubcoreMesh(core_axis_name="c", subcore_axis_name="s",
                                  num_cores=_NC, num_subcores=_NS)
    out_type = [jax.ShapeDtypeStruct((_NW, 8, n), jnp.float32)]
    scratch = [
        pltpu.VMEM((_C,), jnp.int32),
        pltpu.VMEM((_C,), jnp.int32),
        pltpu.VMEM((_C, _HW), jnp.float32),
        pltpu.VMEM((_C, _TW), jnp.float32),
        pltpu.VMEM((8, n), jnp.float32),
    ]
    if with_cnt:
        out_type.append(jax.ShapeDtypeStruct((_NW, n), jnp.float32))
        scratch.append(pltpu.VMEM((n,), jnp.float32))
    scratch.append(pltpu.SemaphoreType.DMA)
    return pl.kernel(
        functools.partial(_edge_body, with_cnt),
        out_type=tuple(out_type),
        mesh=mesh,
        scratch_types=scratch,
        compiler_params=pltpu.CompilerParams(needs_layout_passes=False),
    )(tab, h, src, dst)


def _update_body(*refs, mean, act):
    if mean:
        parts_ref, cnts_ref, xT_ref, rT_ref, b_ref, o_ref = refs
    else:
        parts_ref, xT_ref, rT_ref, b_ref, o_ref = refs
    aggT = jnp.sum(parts_ref[...], axis=0)
    if mean:
        cnt = jnp.sum(cnts_ref[...], axis=0)[None, :]
        aggT = aggT / jnp.maximum(cnt, 1.0)
    y = aggT + jnp.dot(rT_ref[...], xT_ref[...],
                       preferred_element_type=jnp.float32)
    y = y + b_ref[...]
    if act:
        y = jnp.maximum(y, 0.0)
    o_ref[...] = y


def _node_update(parts, cnts, xT, root, bias, mean, act):
    n = xT.shape[1]
    dout = root.shape[1]
    args = [parts] + ([cnts] if mean else []) + [
        xT, root.T, bias.reshape(dout, 1)]
    return pl.pallas_call(
        functools.partial(_update_body, mean=mean, act=act),
        out_shape=jax.ShapeDtypeStruct((dout, n), jnp.float32),
    )(*args)


def _layer(x_in, src, dst, h_edge, p, cnts, mean, act):
    din, dout = p['root'].shape
    w2t = p['w2'].reshape(_ENH, din, dout).transpose(1, 0, 2).reshape(din, _ENH * dout)
    tab = x_in @ w2t
    c = x_in @ p['b2'].reshape(din, dout)
    e = src.shape[0]
    hfull = jnp.concatenate(
        [h_edge, c[src], jnp.zeros((e, _HW - _ENH - dout), jnp.float32)], axis=1)
    if cnts is None and mean:
        parts, cnts = _sc_edge_phase(tab, hfull, src, dst, True)
    else:
        (parts,) = _sc_edge_phase(tab, hfull, src, dst, False)
    yT = _node_update(parts, cnts, x_in.T, p['root'], p['bias'], mean, act)
    return yT.T, cnts


def kernel(x, edge_index, edge_attr, params):
    src = edge_index[0]
    dst = edge_index[1]
    h1e = jax.nn.relu(edge_attr @ params['conv1']['w1'] + params['conv1']['b1'])
    h2e = jax.nn.relu(edge_attr @ params['conv2']['w1'] + params['conv2']['b1'])
    hme = jax.nn.relu(edge_attr @ params['mu']['w1'] + params['mu']['b1'])
    hle = jax.nn.relu(edge_attr @ params['logvar']['w1'] + params['logvar']['b1'])
    h1, cnts = _layer(x, src, dst, h1e, params['conv1'], None, True, True)
    h2, _ = _layer(h1, src, dst, h2e, params['conv2'], cnts, True, True)
    mu, _ = _layer(h2, src, dst, hme, params['mu'], None, False, False)
    lv, _ = _layer(h2, src, dst, hle, params['logvar'], None, False, False)
    return (mu, lv)
